# Initial kernel scaffold; baseline (speedup 1.0000x reference)
import jax
import jax.numpy as jnp
from jax.experimental import pallas as pl


def kernel(x, edge_index, edge_weight, h, c, W_xz, b_xz, W_hz, b_hz, W_xr, b_xr, W_hr, b_hr, W_xh, b_xh, W_hh, b_hh, W_lin, b_lin):
    raise NotImplementedError("write your pallas kernel here")


# speedup vs baseline: 7.0330x; 7.0330x over previous
import functools

import jax
import jax.numpy as jnp
from jax import lax
from jax.experimental import pallas as pl
from jax.experimental.pallas import tpu as pltpu, tpu_sc as plsc

N = 10000
NP = 10240
E = 160000
F_IN = 256
FQ = 64
HID = 128

NC = 2
NS = 16
ET = E // NS
K = 400
NCHUNK = ET // K
NPT = NP // NS


def _sc_sparse(xq, src, dst, ew):
    mesh = plsc.VectorSubcoreMesh(core_axis_name="c", subcore_axis_name="s")

    @functools.partial(
        pl.kernel,
        mesh=mesh,
        compiler_params=pltpu.CompilerParams(
            needs_layout_passes=False, use_tc_tiling_on_sc=False),
        out_type=[
            jax.ShapeDtypeStruct((4 * NP, FQ), jnp.float32),
            jax.ShapeDtypeStruct((NP,), jnp.float32),
        ],
        scratch_types=[
            pltpu.VMEM((ET,), jnp.int32),
            pltpu.VMEM((ET,), jnp.int32),
            pltpu.VMEM((ET,), jnp.float32),
            pltpu.VMEM((NP,), jnp.float32),
            pltpu.VMEM((ET,), jnp.float32),
            pltpu.VMEM((K, FQ), jnp.float32),
            pltpu.VMEM((K,), jnp.int32),
            pltpu.VMEM((3, 128), jnp.int32),
            pltpu.VMEM((16,), jnp.int32),
            pltpu.VMEM((3, 128), jnp.int32),
            pltpu.VMEM((16,), jnp.int32),
            pltpu.VMEM((NPT,), jnp.float32),
            pltpu.VMEM((NPT,), jnp.float32),
            pltpu.VMEM_SHARED((NP, FQ), jnp.float32),
            pltpu.VMEM_SHARED((NP,), jnp.float32),
            pltpu.SemaphoreType.DMA,
        ],
    )
    def k(xq_hbm, src_hbm, dst_hbm, ew_hbm, scat_hbm, dis_hbm,
          src_full, dst_full, ew_full, dis_v, sfull, rows, iabuf,
          dst2, dtail, src2, stail, degsl, dslice, acc, disshared, sem):
        c = lax.axis_index("c")
        s = lax.axis_index("s")
        ebase = s * ET
        nb = s * NPT
        zeros16 = jnp.zeros((16,), jnp.float32)

        pltpu.sync_copy(src_hbm.at[pl.ds(ebase, ET)], src_full)
        pltpu.sync_copy(dst_hbm.at[pl.ds(ebase, ET)], dst_full)
        pltpu.sync_copy(ew_hbm.at[pl.ds(ebase, ET)], ew_full)

        def zds(g, carry):
            dslice[pl.ds(g * 16, 16)] = zeros16
            return carry
        lax.fori_loop(0, NPT // 16, zds, 0)
        pltpu.sync_copy(dslice, disshared.at[pl.ds(nb, NPT)])
        plsc.subcore_barrier()

        def dchunk(t, carry):
            a = t * K
            for g in range(K // 16):
                v = src_full[pl.ds(a + g * 16, 16)]
                if g < 24:
                    src2[g // 8, pl.ds((g % 8) * 16, 16)] = v
                else:
                    stail[...] = v
            for p in range(3):
                pltpu.sync_copy(ew_full.at[pl.ds(a + p * 128, 128)],
                                disshared.at[src2.at[p]], add=True)
            pltpu.sync_copy(ew_full.at[pl.ds(a + 384, 16)],
                            disshared.at[stail], add=True)
            return carry
        lax.fori_loop(0, NCHUNK, dchunk, 0)
        plsc.subcore_barrier()

        pltpu.sync_copy(disshared.at[pl.ds(nb, NPT)], degsl)

        def comb(g, carry):
            col = g * 16
            a = degsl[pl.ds(col, 16)]
            i = lax.bitcast_convert_type(a, jnp.int32)
            i = jnp.int32(0x5F3759DF) - lax.shift_right_logical(i, 1)
            y = lax.bitcast_convert_type(i, jnp.float32)
            for _ in range(3):
                y = y * (1.5 - 0.5 * a * y * y)
            y = jnp.where(a > 0.0, y, 0.0)
            dslice[pl.ds(col, 16)] = y
            return carry
        lax.fori_loop(0, NPT // 16, comb, 0)

        pltpu.sync_copy(dslice, disshared.at[pl.ds(nb, NPT)])

        @pl.when(c == 0)
        def _():
            pltpu.sync_copy(dslice, dis_hbm.at[pl.ds(nb, NPT)])

        plsc.subcore_barrier()
        pltpu.sync_copy(disshared, dis_v)

        def coef(g, carry):
            off = g * 16
            idx16 = src_full[pl.ds(off, 16)]
            d16 = plsc.load_gather(dis_v, [idx16])
            sfull[pl.ds(off, 16)] = ew_full[pl.ds(off, 16)] * d16
            return carry
        lax.fori_loop(0, ET // 16, coef, 0)

        def zrows(j, carry):
            for i in range(FQ // 16):
                rows[j, pl.ds(i * 16, 16)] = zeros16
            return carry
        lax.fori_loop(0, K, zrows, 0)

        for q in range(2):
            qq = 2 * c + q
            roff = qq * NP

            pltpu.sync_copy(rows, acc.at[pl.ds(nb, K)])
            pltpu.sync_copy(rows.at[pl.ds(0, NPT - K)],
                            acc.at[pl.ds(nb + K, NPT - K)])
            plsc.subcore_barrier()

            def chunk(t, carry):
                a = t * K
                for g in range(K // 16):
                    off = a + g * 16
                    idx16 = src_full[pl.ds(off, 16)]
                    iabuf[pl.ds(g * 16, 16)] = idx16 + roff
                    dv = dst_full[pl.ds(off, 16)]
                    if g < 24:
                        dst2[g // 8, pl.ds((g % 8) * 16, 16)] = dv
                    else:
                        dtail[...] = dv
                pltpu.async_copy(xq_hbm.at[iabuf], rows, sem).wait()

                def scale(g2, carry2):
                    base = g2 * 16
                    s16 = sfull[pl.ds(a + base, 16)]
                    for jj in range(16):
                        sj = s16[jj]
                        j = base + jj
                        for i in range(FQ // 16):
                            rows[j, pl.ds(i * 16, 16)] = (
                                rows[j, pl.ds(i * 16, 16)] * sj)
                    return carry2
                lax.fori_loop(0, K // 16, scale, 0)

                for p in range(3):
                    pltpu.sync_copy(rows.at[pl.ds(p * 128, 128)],
                                    acc.at[dst2.at[p]], add=True)
                pltpu.sync_copy(rows.at[pl.ds(384, 16)], acc.at[dtail],
                                add=True)
                return carry
            lax.fori_loop(0, NCHUNK, chunk, 0)

            plsc.subcore_barrier()
            pltpu.sync_copy(acc.at[pl.ds(nb, NPT)],
                            scat_hbm.at[pl.ds(roff + nb, NPT)])
            if q == 0:
                def zrows2(j, carry):
                    for i in range(FQ // 16):
                        rows[j, pl.ds(i * 16, 16)] = zeros16
                    return carry
                lax.fori_loop(0, K, zrows2, 0)

    return k(xq, src, dst, ew)


def _tc_dense_body(x_ref, s0_ref, s1_ref, s2_ref, s3_ref, dis_ref, wx_ref,
                   wt0_ref, wt1_ref, wt2_ref, wt3_ref, b_ref, wl_ref, bl_ref,
                   out_ref):
    xb = x_ref[...]
    ndis = -dis_ref[...]
    g = (jnp.dot(xb, wx_ref[...], preferred_element_type=jnp.float32)
         + jnp.dot(ndis * s0_ref[...], wt0_ref[...],
                   preferred_element_type=jnp.float32)
         + jnp.dot(ndis * s1_ref[...], wt1_ref[...],
                   preferred_element_type=jnp.float32)
         + jnp.dot(ndis * s2_ref[...], wt2_ref[...],
                   preferred_element_type=jnp.float32)
         + jnp.dot(ndis * s3_ref[...], wt3_ref[...],
                   preferred_element_type=jnp.float32)
         + b_ref[...])
    z = jax.nn.sigmoid(g[:, :HID])
    ht = jnp.tanh(g[:, HID:])
    h = (1.0 - z) * ht
    o = jnp.dot(jax.nn.relu(h), wl_ref[...],
                preferred_element_type=jnp.float32) + bl_ref[...]
    out_ref[...] = o


def _tc_dense(x, squads, dis2, wx, wtq, bvec, wl, bl2):
    bm = 400
    grid = (N // bm,)
    return pl.pallas_call(
        _tc_dense_body,
        grid=grid,
        in_specs=[
            pl.BlockSpec((bm, F_IN), lambda i: (i, 0)),
            pl.BlockSpec((bm, FQ), lambda i: (i, 0)),
            pl.BlockSpec((bm, FQ), lambda i: (i, 0)),
            pl.BlockSpec((bm, FQ), lambda i: (i, 0)),
            pl.BlockSpec((bm, FQ), lambda i: (i, 0)),
            pl.BlockSpec((bm, 1), lambda i: (i, 0)),
            pl.BlockSpec((F_IN, 2 * HID), lambda i: (0, 0)),
            pl.BlockSpec((FQ, 2 * HID), lambda i: (0, 0)),
            pl.BlockSpec((FQ, 2 * HID), lambda i: (0, 0)),
            pl.BlockSpec((FQ, 2 * HID), lambda i: (0, 0)),
            pl.BlockSpec((FQ, 2 * HID), lambda i: (0, 0)),
            pl.BlockSpec((1, 2 * HID), lambda i: (0, 0)),
            pl.BlockSpec((HID, 1), lambda i: (0, 0)),
            pl.BlockSpec((1, 1), lambda i: (0, 0)),
        ],
        out_specs=pl.BlockSpec((bm, 1), lambda i: (i, 0)),
        out_shape=jax.ShapeDtypeStruct((N, 1), jnp.float32),
    )(x, *squads, dis2, wx, *wtq, bvec, wl, bl2)


def kernel(x, edge_index, edge_weight, h, c,
           W_xz, b_xz, W_hz, b_hz, W_xr, b_xr, W_hr, b_hr,
           W_xh, b_xh, W_hh, b_hh, W_lin, b_lin):
    src = edge_index[0]
    dst = edge_index[1]

    xp = jnp.pad(x, ((0, NP - N), (0, 0)))
    xq = jnp.concatenate(
        [xp[:, 0:64], xp[:, 64:128], xp[:, 128:192], xp[:, 192:256]], axis=0)

    scat, dis = _sc_sparse(xq, src, dst, edge_weight)

    squads = [scat[q * NP:q * NP + N] for q in range(4)]
    dis2 = dis[:N, None]

    wx = jnp.concatenate([W_xz[0], W_xh[0]], axis=1)
    wt = jnp.concatenate([W_xz[1], W_xh[1]], axis=1)
    wtq = [wt[q * FQ:(q + 1) * FQ] for q in range(4)]
    bvec = jnp.concatenate([b_xz + b_hz, b_xh + b_hh])[None, :]
    bl2 = b_lin[None, :]

    return _tc_dense(x, squads, dis2, wx, wtq, bvec, W_lin, bl2)


# speedup vs baseline: 8.0286x; 1.1416x over previous
import functools

import jax
import jax.numpy as jnp
from jax import lax
from jax.experimental import pallas as pl
from jax.experimental.pallas import tpu as pltpu, tpu_sc as plsc

N = 10000
NP = 10240
E = 160000
F_IN = 256
FQ = 64
HID = 128

NC = 2
NS = 16
ET = E // NS
K = 80
NCHUNK = ET // K
KD = 400
NDCHUNK = ET // KD
NPT = NP // NS


def _sc_sparse(xq, src, dst, ew):
    mesh = plsc.VectorSubcoreMesh(core_axis_name="c", subcore_axis_name="s")

    @functools.partial(
        pl.kernel,
        mesh=mesh,
        compiler_params=pltpu.CompilerParams(
            needs_layout_passes=False, use_tc_tiling_on_sc=False),
        out_type=[
            jax.ShapeDtypeStruct((4 * NP, FQ), jnp.float32),
            jax.ShapeDtypeStruct((NP,), jnp.float32),
        ],
        scratch_types=[
            pltpu.VMEM((ET,), jnp.int32),
            pltpu.VMEM((ET,), jnp.int32),
            pltpu.VMEM((ET,), jnp.float32),
            pltpu.VMEM((NP,), jnp.float32),
            pltpu.VMEM((ET,), jnp.float32),
            pltpu.VMEM((K, FQ), jnp.float32),
            pltpu.VMEM((K, FQ), jnp.float32),
            pltpu.VMEM((K,), jnp.int32),
            pltpu.VMEM((K,), jnp.int32),
            pltpu.VMEM((K,), jnp.int32),
            pltpu.VMEM((3, 128), jnp.int32),
            pltpu.VMEM((16,), jnp.int32),
            pltpu.VMEM((NPT,), jnp.float32),
            pltpu.VMEM((NPT,), jnp.float32),
            pltpu.VMEM_SHARED((NP, FQ), jnp.float32),
            pltpu.VMEM_SHARED((NP,), jnp.float32),
            pltpu.SemaphoreType.DMA,
            pltpu.SemaphoreType.DMA,
        ],
    )
    def k(xq_hbm, src_hbm, dst_hbm, ew_hbm, scat_hbm, dis_hbm,
          src_full, dst_full, ew_full, dis_v, sfull, rows0, rows1, ia0, ia1,
          dstc, src2, stail, degsl, dslice, acc, disshared,
          sem0, sem1):
        c = lax.axis_index("c")
        s = lax.axis_index("s")
        ebase = s * ET
        nb = s * NPT
        zeros16 = jnp.zeros((16,), jnp.float32)

        pltpu.sync_copy(src_hbm.at[pl.ds(ebase, ET)], src_full)
        pltpu.sync_copy(dst_hbm.at[pl.ds(ebase, ET)], dst_full)
        pltpu.sync_copy(ew_hbm.at[pl.ds(ebase, ET)], ew_full)

        def zds(g, carry):
            dslice[pl.ds(g * 16, 16)] = zeros16
            return carry
        lax.fori_loop(0, NPT // 16, zds, 0)
        pltpu.sync_copy(dslice, disshared.at[pl.ds(nb, NPT)])
        plsc.subcore_barrier()

        def dchunk(t, carry):
            a = t * KD
            for g in range(KD // 16):
                v = src_full[pl.ds(a + g * 16, 16)]
                if g < 24:
                    src2[g // 8, pl.ds((g % 8) * 16, 16)] = v
                else:
                    stail[...] = v
            for p in range(3):
                pltpu.sync_copy(ew_full.at[pl.ds(a + p * 128, 128)],
                                disshared.at[src2.at[p]], add=True)
            pltpu.sync_copy(ew_full.at[pl.ds(a + 384, 16)],
                            disshared.at[stail], add=True)
            return carry
        lax.fori_loop(0, NDCHUNK, dchunk, 0)
        plsc.subcore_barrier()

        pltpu.sync_copy(disshared.at[pl.ds(nb, NPT)], degsl)

        def comb(g, carry):
            col = g * 16
            a = degsl[pl.ds(col, 16)]
            i = lax.bitcast_convert_type(a, jnp.int32)
            i = jnp.int32(0x5F3759DF) - lax.shift_right_logical(i, 1)
            y = lax.bitcast_convert_type(i, jnp.float32)
            for _ in range(3):
                y = y * (1.5 - 0.5 * a * y * y)
            y = jnp.where(a > 0.0, y, 0.0)
            dslice[pl.ds(col, 16)] = y
            return carry
        lax.fori_loop(0, NPT // 16, comb, 0)

        pltpu.sync_copy(dslice, disshared.at[pl.ds(nb, NPT)])

        @pl.when(c == 0)
        def _():
            pltpu.sync_copy(dslice, dis_hbm.at[pl.ds(nb, NPT)])

        plsc.subcore_barrier()
        pltpu.sync_copy(disshared, dis_v)

        def coef(g, carry):
            off = g * 16
            idx16 = src_full[pl.ds(off, 16)]
            d16 = plsc.load_gather(dis_v, [idx16])
            sfull[pl.ds(off, 16)] = ew_full[pl.ds(off, 16)] * d16
            return carry
        lax.fori_loop(0, ET // 16, coef, 0)

        def zrows(j, carry):
            for i in range(FQ // 16):
                rows0[j, pl.ds(i * 16, 16)] = zeros16
            return carry
        lax.fori_loop(0, K, zrows, 0)

        for q in range(2):
            qq = 2 * c + q
            roff = qq * NP

            for r in range(NPT // K):
                pltpu.sync_copy(rows0, acc.at[pl.ds(nb + r * K, K)])
            plsc.subcore_barrier()

            def fill_ia(t, iab):
                a = t * K
                for g in range(K // 16):
                    idx16 = src_full[pl.ds(a + g * 16, 16)]
                    iab[pl.ds(g * 16, 16)] = idx16 + roff

            def consume(t, rows):
                a = t * K

                def scale(g2, carry2):
                    base = g2 * 16
                    s16 = sfull[pl.ds(a + base, 16)]
                    for jj in range(16):
                        sj = s16[jj]
                        j = base + jj
                        for i in range(FQ // 16):
                            rows[j, pl.ds(i * 16, 16)] = (
                                rows[j, pl.ds(i * 16, 16)] * sj)
                    return carry2
                lax.fori_loop(0, K // 16, scale, 0)

                for g in range(K // 16):
                    dv = dst_full[pl.ds(a + g * 16, 16)]
                    dstc[pl.ds(g * 16, 16)] = dv
                pltpu.sync_copy(rows, acc.at[dstc], add=True)

            fill_ia(0, ia0)
            pltpu.async_copy(xq_hbm.at[ia0], rows0, sem0)

            def pair(u, carry):
                t0 = u * 2
                fill_ia(t0 + 1, ia1)
                pltpu.async_copy(xq_hbm.at[ia1], rows1, sem1)
                pltpu.make_async_copy(xq_hbm.at[ia0], rows0, sem0).wait()
                consume(t0, rows0)
                fill_ia(t0 + 2, ia0)
                pltpu.async_copy(xq_hbm.at[ia0], rows0, sem0)
                pltpu.make_async_copy(xq_hbm.at[ia1], rows1, sem1).wait()
                consume(t0 + 1, rows1)
                return carry
            lax.fori_loop(0, (NCHUNK - 1) // 2, pair, 0)

            pltpu.make_async_copy(xq_hbm.at[ia0], rows0, sem0).wait()
            consume(NCHUNK - 1, rows0)

            plsc.subcore_barrier()
            pltpu.sync_copy(acc.at[pl.ds(nb, NPT)],
                            scat_hbm.at[pl.ds(roff + nb, NPT)])
            if q == 0:
                def zrows2(j, carry):
                    for i in range(FQ // 16):
                        rows0[j, pl.ds(i * 16, 16)] = zeros16
                    return carry
                lax.fori_loop(0, K, zrows2, 0)

    return k(xq, src, dst, ew)


def _tc_dense_body(x_ref, s0_ref, s1_ref, s2_ref, s3_ref, dis_ref, wx_ref,
                   wt0_ref, wt1_ref, wt2_ref, wt3_ref, b_ref, wl_ref, bl_ref,
                   out_ref):
    xb = x_ref[...]
    ndis = -dis_ref[...]
    g = (jnp.dot(xb, wx_ref[...], preferred_element_type=jnp.float32)
         + jnp.dot(ndis * s0_ref[...], wt0_ref[...],
                   preferred_element_type=jnp.float32)
         + jnp.dot(ndis * s1_ref[...], wt1_ref[...],
                   preferred_element_type=jnp.float32)
         + jnp.dot(ndis * s2_ref[...], wt2_ref[...],
                   preferred_element_type=jnp.float32)
         + jnp.dot(ndis * s3_ref[...], wt3_ref[...],
                   preferred_element_type=jnp.float32)
         + b_ref[...])
    z = jax.nn.sigmoid(g[:, :HID])
    ht = jnp.tanh(g[:, HID:])
    h = (1.0 - z) * ht
    o = jnp.dot(jax.nn.relu(h), wl_ref[...],
                preferred_element_type=jnp.float32) + bl_ref[...]
    out_ref[...] = o


def _tc_dense(x, squads, dis2, wx, wtq, bvec, wl, bl2):
    bm = 400
    grid = (N // bm,)
    return pl.pallas_call(
        _tc_dense_body,
        grid=grid,
        in_specs=[
            pl.BlockSpec((bm, F_IN), lambda i: (i, 0)),
            pl.BlockSpec((bm, FQ), lambda i: (i, 0)),
            pl.BlockSpec((bm, FQ), lambda i: (i, 0)),
            pl.BlockSpec((bm, FQ), lambda i: (i, 0)),
            pl.BlockSpec((bm, FQ), lambda i: (i, 0)),
            pl.BlockSpec((bm, 1), lambda i: (i, 0)),
            pl.BlockSpec((F_IN, 2 * HID), lambda i: (0, 0)),
            pl.BlockSpec((FQ, 2 * HID), lambda i: (0, 0)),
            pl.BlockSpec((FQ, 2 * HID), lambda i: (0, 0)),
            pl.BlockSpec((FQ, 2 * HID), lambda i: (0, 0)),
            pl.BlockSpec((FQ, 2 * HID), lambda i: (0, 0)),
            pl.BlockSpec((1, 2 * HID), lambda i: (0, 0)),
            pl.BlockSpec((HID, 1), lambda i: (0, 0)),
            pl.BlockSpec((1, 1), lambda i: (0, 0)),
        ],
        out_specs=pl.BlockSpec((bm, 1), lambda i: (i, 0)),
        out_shape=jax.ShapeDtypeStruct((N, 1), jnp.float32),
    )(x, *squads, dis2, wx, *wtq, bvec, wl, bl2)


def kernel(x, edge_index, edge_weight, h, c,
           W_xz, b_xz, W_hz, b_hz, W_xr, b_xr, W_hr, b_hr,
           W_xh, b_xh, W_hh, b_hh, W_lin, b_lin):
    src = edge_index[0]
    dst = edge_index[1]

    xp = jnp.pad(x, ((0, NP - N), (0, 0)))
    xq = jnp.concatenate(
        [xp[:, 0:64], xp[:, 64:128], xp[:, 128:192], xp[:, 192:256]], axis=0)

    scat, dis = _sc_sparse(xq, src, dst, edge_weight)

    squads = [scat[q * NP:q * NP + N] for q in range(4)]
    dis2 = dis[:N, None]

    wx = jnp.concatenate([W_xz[0], W_xh[0]], axis=1)
    wt = jnp.concatenate([W_xz[1], W_xh[1]], axis=1)
    wtq = [wt[q * FQ:(q + 1) * FQ] for q in range(4)]
    bvec = jnp.concatenate([b_xz + b_hz, b_xh + b_hh])[None, :]
    bl2 = b_lin[None, :]

    return _tc_dense(x, squads, dis2, wx, wtq, bvec, W_lin, bl2)


# speedup vs baseline: 18.2897x; 2.2781x over previous
import functools

import jax
import jax.numpy as jnp
from jax import lax
from jax.experimental import pallas as pl
from jax.experimental.pallas import tpu as pltpu, tpu_sc as plsc

N = 10000
NP = 10240
E = 160000
F_IN = 256
FH = 128
HID = 128

NC = 2
NS = 16
ET = E // NS
K = 80
NCHUNK = ET // K
KD = 400
NDCHUNK = ET // KD
NPT = NP // NS


def _sc_sparse(xh, src, dst, ew):
    mesh = plsc.VectorSubcoreMesh(core_axis_name="c", subcore_axis_name="s")

    @functools.partial(
        pl.kernel,
        mesh=mesh,
        compiler_params=pltpu.CompilerParams(
            needs_layout_passes=False, use_tc_tiling_on_sc=False),
        out_type=[
            jax.ShapeDtypeStruct((2 * NP, FH), jnp.bfloat16),
            jax.ShapeDtypeStruct((NP,), jnp.float32),
        ],
        scratch_types=[
            pltpu.VMEM((ET,), jnp.int32),
            pltpu.VMEM((ET,), jnp.int32),
            pltpu.VMEM((ET,), jnp.float32),
            pltpu.VMEM((NP,), jnp.float32),
            pltpu.VMEM((ET,), jnp.float32),
            pltpu.VMEM((K, FH), jnp.bfloat16),
            pltpu.VMEM((K, FH), jnp.bfloat16),
            pltpu.VMEM((K,), jnp.int32),
            pltpu.VMEM((K,), jnp.int32),
            pltpu.VMEM((K,), jnp.int32),
            pltpu.VMEM((3, 128), jnp.int32),
            pltpu.VMEM((16,), jnp.int32),
            pltpu.VMEM((NPT,), jnp.float32),
            pltpu.VMEM((NPT,), jnp.float32),
            pltpu.VMEM_SHARED((NP, FH), jnp.bfloat16),
            pltpu.VMEM_SHARED((NP,), jnp.float32),
            pltpu.SemaphoreType.DMA,
            pltpu.SemaphoreType.DMA,
        ],
    )
    def k(xh_hbm, src_hbm, dst_hbm, ew_hbm, scat_hbm, dis_hbm,
          src_full, dst_full, ew_full, dis_v, sfull, rows0, rows1, ia0, ia1,
          dstc, src2, stail, degsl, dslice, acc, disshared,
          sem0, sem1):
        c = lax.axis_index("c")
        s = lax.axis_index("s")
        ebase = s * ET
        nb = s * NPT
        zeros16 = jnp.zeros((16,), jnp.float32)
        zeros32b = jnp.zeros((32,), jnp.bfloat16)

        pltpu.sync_copy(src_hbm.at[pl.ds(ebase, ET)], src_full)
        pltpu.sync_copy(dst_hbm.at[pl.ds(ebase, ET)], dst_full)
        pltpu.sync_copy(ew_hbm.at[pl.ds(ebase, ET)], ew_full)

        def zds(g, carry):
            dslice[pl.ds(g * 16, 16)] = zeros16
            return carry
        lax.fori_loop(0, NPT // 16, zds, 0)
        pltpu.sync_copy(dslice, disshared.at[pl.ds(nb, NPT)])
        plsc.subcore_barrier()

        def dchunk(t, carry):
            a = t * KD
            for g in range(KD // 16):
                v = src_full[pl.ds(a + g * 16, 16)]
                if g < 24:
                    src2[g // 8, pl.ds((g % 8) * 16, 16)] = v
                else:
                    stail[...] = v
            for p in range(3):
                pltpu.sync_copy(ew_full.at[pl.ds(a + p * 128, 128)],
                                disshared.at[src2.at[p]], add=True)
            pltpu.sync_copy(ew_full.at[pl.ds(a + 384, 16)],
                            disshared.at[stail], add=True)
            return carry
        lax.fori_loop(0, NDCHUNK, dchunk, 0)
        plsc.subcore_barrier()

        pltpu.sync_copy(disshared.at[pl.ds(nb, NPT)], degsl)

        def comb(g, carry):
            col = g * 16
            a = degsl[pl.ds(col, 16)]
            i = lax.bitcast_convert_type(a, jnp.int32)
            i = jnp.int32(0x5F3759DF) - lax.shift_right_logical(i, 1)
            y = lax.bitcast_convert_type(i, jnp.float32)
            for _ in range(3):
                y = y * (1.5 - 0.5 * a * y * y)
            y = jnp.where(a > 0.0, y, 0.0)
            dslice[pl.ds(col, 16)] = y
            return carry
        lax.fori_loop(0, NPT // 16, comb, 0)

        pltpu.sync_copy(dslice, disshared.at[pl.ds(nb, NPT)])

        @pl.when(c == 0)
        def _():
            pltpu.sync_copy(dslice, dis_hbm.at[pl.ds(nb, NPT)])

        plsc.subcore_barrier()
        pltpu.sync_copy(disshared, dis_v)

        def coef(g, carry):
            off = g * 16
            idx16 = src_full[pl.ds(off, 16)]
            d16 = plsc.load_gather(dis_v, [idx16])
            sfull[pl.ds(off, 16)] = ew_full[pl.ds(off, 16)] * d16
            return carry
        lax.fori_loop(0, ET // 16, coef, 0)

        def zrows(j, carry):
            for i in range(FH // 32):
                rows0[j, pl.ds(i * 32, 32)] = zeros32b
            return carry
        lax.fori_loop(0, K, zrows, 0)

        roff = c * NP

        for r in range(NPT // K):
            pltpu.sync_copy(rows0, acc.at[pl.ds(nb + r * K, K)])
        plsc.subcore_barrier()

        def fill_ia(t, iab):
            a = t * K
            for g in range(K // 16):
                idx16 = src_full[pl.ds(a + g * 16, 16)]
                iab[pl.ds(g * 16, 16)] = idx16 + roff

        def consume(t, rows):
            a = t * K

            def scale(g2, carry2):
                base = g2 * 16
                s16 = sfull[pl.ds(a + base, 16)]
                for jj in range(16):
                    sj = s16[jj]
                    j = base + jj
                    for i in range(FH // 32):
                        v = rows[j, pl.ds(i * 32, 32)]
                        va, vb = plsc.unpack(
                            v, format=plsc.PackFormat.INTERLEAVED)
                        rows[j, pl.ds(i * 32, 32)] = plsc.pack(
                            va * sj, vb * sj,
                            format=plsc.PackFormat.INTERLEAVED)
                return carry2
            lax.fori_loop(0, K // 16, scale, 0)

            for g in range(K // 16):
                dv = dst_full[pl.ds(a + g * 16, 16)]
                dstc[pl.ds(g * 16, 16)] = dv
            pltpu.sync_copy(rows, acc.at[dstc], add=True)

        fill_ia(0, ia0)
        pltpu.async_copy(xh_hbm.at[ia0], rows0, sem0)

        def pair(u, carry):
            t0 = u * 2
            fill_ia(t0 + 1, ia1)
            pltpu.async_copy(xh_hbm.at[ia1], rows1, sem1)
            pltpu.make_async_copy(xh_hbm.at[ia0], rows0, sem0).wait()
            consume(t0, rows0)
            fill_ia(t0 + 2, ia0)
            pltpu.async_copy(xh_hbm.at[ia0], rows0, sem0)
            pltpu.make_async_copy(xh_hbm.at[ia1], rows1, sem1).wait()
            consume(t0 + 1, rows1)
            return carry
        lax.fori_loop(0, (NCHUNK - 1) // 2, pair, 0)

        pltpu.make_async_copy(xh_hbm.at[ia0], rows0, sem0).wait()
        consume(NCHUNK - 1, rows0)

        plsc.subcore_barrier()
        pltpu.sync_copy(acc.at[pl.ds(nb, NPT)],
                        scat_hbm.at[pl.ds(c * NP + nb, NPT)])

    return k(xh, src, dst, ew)


def _tc_dense_body(x_ref, s0_ref, s1_ref, dis_ref, wx_ref, wt0_ref, wt1_ref,
                   b_ref, wl_ref, bl_ref, out_ref):
    xb = x_ref[...]
    ndis = -dis_ref[...]
    t0 = (ndis * s0_ref[...].astype(jnp.float32)).astype(jnp.bfloat16)
    t1 = (ndis * s1_ref[...].astype(jnp.float32)).astype(jnp.bfloat16)
    g = (jnp.dot(xb, wx_ref[...], preferred_element_type=jnp.float32)
         + jnp.dot(t0, wt0_ref[...], preferred_element_type=jnp.float32)
         + jnp.dot(t1, wt1_ref[...], preferred_element_type=jnp.float32)
         + b_ref[...])
    z = jax.nn.sigmoid(g[:, :HID])
    ht = jnp.tanh(g[:, HID:])
    h = (1.0 - z) * ht
    o = jnp.dot(jax.nn.relu(h), wl_ref[...],
                preferred_element_type=jnp.float32) + bl_ref[...]
    out_ref[...] = o


def _tc_dense(x, s0, s1, dis2, wx, wt0, wt1, bvec, wl, bl2):
    bm = 400
    grid = (N // bm,)
    return pl.pallas_call(
        _tc_dense_body,
        grid=grid,
        in_specs=[
            pl.BlockSpec((bm, F_IN), lambda i: (i, 0)),
            pl.BlockSpec((bm, FH), lambda i: (i, 0)),
            pl.BlockSpec((bm, FH), lambda i: (i, 0)),
            pl.BlockSpec((bm, 1), lambda i: (i, 0)),
            pl.BlockSpec((F_IN, 2 * HID), lambda i: (0, 0)),
            pl.BlockSpec((FH, 2 * HID), lambda i: (0, 0)),
            pl.BlockSpec((FH, 2 * HID), lambda i: (0, 0)),
            pl.BlockSpec((1, 2 * HID), lambda i: (0, 0)),
            pl.BlockSpec((HID, 1), lambda i: (0, 0)),
            pl.BlockSpec((1, 1), lambda i: (0, 0)),
        ],
        out_specs=pl.BlockSpec((bm, 1), lambda i: (i, 0)),
        out_shape=jax.ShapeDtypeStruct((N, 1), jnp.float32),
    )(x, s0, s1, dis2, wx, wt0, wt1, bvec, wl, bl2)


def kernel(x, edge_index, edge_weight, h, c,
           W_xz, b_xz, W_hz, b_hz, W_xr, b_xr, W_hr, b_hr,
           W_xh, b_xh, W_hh, b_hh, W_lin, b_lin):
    src = edge_index[0]
    dst = edge_index[1]

    xb16 = x.astype(jnp.bfloat16)
    xp = jnp.pad(xb16, ((0, NP - N), (0, 0)))
    xh = jnp.concatenate([xp[:, :FH], xp[:, FH:]], axis=0)

    scat, dis = _sc_sparse(xh, src, dst, edge_weight)

    s0 = scat[:N]
    s1 = scat[NP:NP + N]
    dis2 = dis[:N, None]

    wx = jnp.concatenate([W_xz[0], W_xh[0]], axis=1).astype(jnp.bfloat16)
    wt = jnp.concatenate([W_xz[1], W_xh[1]], axis=1)
    wt0 = wt[:FH].astype(jnp.bfloat16)
    wt1 = wt[FH:].astype(jnp.bfloat16)
    bvec = jnp.concatenate([b_xz + b_hz, b_xh + b_hh])[None, :]
    bl2 = b_lin[None, :]

    return _tc_dense(xb16[:N], s0, s1, dis2, wx, wt0, wt1, bvec, W_lin, bl2)
